# Initial kernel scaffold; baseline (speedup 1.0000x reference)
import jax
import jax.numpy as jnp
from jax.experimental import pallas as pl


def kernel(x, Wr, br, W1, b1, W2, b2, Ws1, bs1, Ws2, bs2):
    raise NotImplementedError("write your pallas kernel here")


# speedup vs baseline: 2.2932x; 2.2932x over previous
import functools
import math

import jax
import jax.numpy as jnp
from jax.experimental import pallas as pl
from jax.experimental.pallas import tpu as pltpu

_SQRT2 = math.sqrt(2.0)


def _gelu(h):
    return 0.5 * h * (1.0 + jax.lax.erf(h / _SQRT2))


def _shared_body(x_ref, w1_ref, b1_ref, w2_ref, b2_ref, out_ref, *, n_shared):
    p = pl.program_id(0)
    j = pl.program_id(1)

    @pl.when((p == 0) & (j == 0))
    def _():
        out_ref[...] = jnp.zeros_like(out_ref)

    h = x_ref[...] @ w1_ref[0] + b1_ref[0, 0]
    h = _gelu(h)
    y = h @ w2_ref[0]
    inv = 1.0 / n_shared

    @pl.when(j == 0)
    def _():
        out_ref[...] += b2_ref[0, 0] * inv

    out_ref[...] += y * inv


def _routed_body(x_ref, wr_ref, br_ref, w1_ref, b1_ref, w2_ref, b2_ref,
                 init_ref, out_ref, wts_ref, *, n_exp):
    p = pl.program_id(0)
    j = pl.program_id(1)

    @pl.when((p == 0) & (j == 0))
    def _():
        gates = x_ref[...] @ wr_ref[...] + br_ref[...]
        lanes = jax.lax.broadcasted_iota(jnp.int32, gates.shape, 1)
        neg = jnp.float32(-1e30)
        big = jnp.int32(2**30)
        m1 = jnp.max(gates, axis=1, keepdims=True)
        i1 = jnp.min(jnp.where(gates == m1, lanes, big), axis=1, keepdims=True)
        g2 = jnp.where(lanes == i1, neg, gates)
        m2 = jnp.max(g2, axis=1, keepdims=True)
        i2 = jnp.min(jnp.where(g2 == m2, lanes, big), axis=1, keepdims=True)
        p1 = 1.0 / (1.0 + jnp.exp(m2 - m1))
        p2 = 1.0 - p1
        wts_ref[...] = (jnp.where(lanes == i1, p1, 0.0)
                        + jnp.where(lanes == i2, p2, 0.0))
        out_ref[...] = init_ref[...]

    h = x_ref[...] @ w1_ref[0] + b1_ref[0, 0]
    h = _gelu(h)
    y = h @ w2_ref[0]
    sel = (jax.lax.broadcasted_iota(jnp.int32, (n_exp, 1), 0) == p
           ).astype(jnp.float32)
    wcol = wts_ref[...] @ sel

    @pl.when(j == 0)
    def _():
        out_ref[...] += b2_ref[0, 0] * wcol

    out_ref[...] += y * wcol


def kernel(x, Wr, br, W1, b1, W2, b2, Ws1, bs1, Ws2, bs2):
    Bb, Ll, Dd = x.shape
    Ee, _, Hh = W1.shape
    Ss = Ws1.shape[0]
    x2 = x.reshape(Ll, Dd)
    br2 = br.reshape(1, Ee)
    b1r = b1.reshape(Ee, 1, Hh)
    b2r = b2.reshape(Ee, 1, Dd)
    bs1r = bs1.reshape(Ss, 1, Hh)
    bs2r = bs2.reshape(Ss, 1, Dd)
    hc = min(Hh, 512)
    jn = Hh // hc

    shared = pl.pallas_call(
        functools.partial(_shared_body, n_shared=Ss),
        grid=(Ss, jn),
        in_specs=[
            pl.BlockSpec((Ll, Dd), lambda p, j: (0, 0)),
            pl.BlockSpec((1, Dd, hc), lambda p, j: (p, 0, j)),
            pl.BlockSpec((1, 1, hc), lambda p, j: (p, 0, j)),
            pl.BlockSpec((1, hc, Dd), lambda p, j: (p, j, 0)),
            pl.BlockSpec((1, 1, Dd), lambda p, j: (p, 0, 0)),
        ],
        out_specs=pl.BlockSpec((Ll, Dd), lambda p, j: (0, 0)),
        out_shape=jax.ShapeDtypeStruct((Ll, Dd), jnp.float32),
    )(x2, Ws1, bs1r, Ws2, bs2r)

    out = pl.pallas_call(
        functools.partial(_routed_body, n_exp=Ee),
        grid=(Ee, jn),
        in_specs=[
            pl.BlockSpec((Ll, Dd), lambda p, j: (0, 0)),
            pl.BlockSpec((Dd, Ee), lambda p, j: (0, 0)),
            pl.BlockSpec((1, Ee), lambda p, j: (0, 0)),
            pl.BlockSpec((1, Dd, hc), lambda p, j: (p, 0, j)),
            pl.BlockSpec((1, 1, hc), lambda p, j: (p, 0, j)),
            pl.BlockSpec((1, hc, Dd), lambda p, j: (p, j, 0)),
            pl.BlockSpec((1, 1, Dd), lambda p, j: (p, 0, 0)),
            pl.BlockSpec((Ll, Dd), lambda p, j: (0, 0)),
        ],
        out_specs=pl.BlockSpec((Ll, Dd), lambda p, j: (0, 0)),
        out_shape=jax.ShapeDtypeStruct((Ll, Dd), jnp.float32),
        scratch_shapes=[pltpu.VMEM((Ll, Ee), jnp.float32)],
    )(x2, Wr, br2, W1, b1r, W2, b2r, shared)

    return out.reshape(Bb, Ll, Dd)
